# baseline (speedup 1.0000x reference)
import functools

import jax
import jax.numpy as jnp
from jax import lax
from jax.experimental import pallas as pl
from jax.experimental.pallas import tpu as pltpu
from jax.experimental.pallas import tpu_sc as plsc

B = 16384
D = 32
U = 1000000
PACK = 4
TCH = 2048
NB = 123
QS = NB * TCH
NBT = (U + TCH - 1) // TCH - 1
NC = 2
NS = 16
NW = NC * NS
BPW = B // NW
CH = 32
NCHUNK = BPW // CH
L = 16
GPC = CH // L


def _tr_body(*refs):
    in_refs, out_refs = refs[:8], refs[8:]
    for t in range(2):
        pieces = [
            jnp.swapaxes(in_refs[4 * t + k][...], 0, 1) for k in range(PACK)
        ]
        out_refs[t][...] = jnp.concatenate(pieces, axis=1)


def _tr_in_spec(k):
    return pl.BlockSpec(
        (D, TCH), lambda i, k=k: (0, jnp.minimum(k * NB + i, NBT)))


_tc_relayout = pl.pallas_call(
    _tr_body,
    grid=(NB,),
    in_specs=[_tr_in_spec(k) for _ in range(2) for k in range(PACK)],
    out_specs=[pl.BlockSpec((TCH, PACK * D), lambda i: (i, 0))] * 2,
    out_shape=[jax.ShapeDtypeStruct((QS, PACK * D), jnp.float32)] * 2,
)

_mesh = plsc.VectorSubcoreMesh(
    core_axis_name="c", subcore_axis_name="s", num_cores=NC, num_subcores=NS
)


RW = 128
NWIN = QS // RW
TAILQ = 3 * QS - (U - 64) - RW * 0
TAILQ0 = U - 64 - 3 * QS + 0
TAILW = U - 64


@functools.partial(
    pl.kernel,
    out_type=(
        jax.ShapeDtypeStruct((QS, PACK * D), jnp.float32),
        jax.ShapeDtypeStruct((QS, PACK * D), jnp.float32),
    ),
    mesh=_mesh,
    scratch_types=[
        pltpu.VMEM((PACK, D, RW), jnp.float32),
        pltpu.VMEM((PACK, D, RW), jnp.float32),
        pltpu.VMEM((RW, PACK * D), jnp.float32),
        pltpu.VMEM((RW, PACK * D), jnp.float32),
        pltpu.SemaphoreType.DMA,
        pltpu.SemaphoreType.DMA,
    ],
    compiler_params=pltpu.CompilerParams(needs_layout_passes=False),
)
def _sc_relayout(guT, giT, tailu, taili, pgu, pgi,
                 gu_b, gi_b, gu_o, gi_o, sem0, sem1):
    wid = lax.axis_index("s") * NC + lax.axis_index("c")

    def win_body(n, carry):
        m = n * NW + wid

        @pl.when(m < NWIN)
        def _():
            q0 = pl.multiple_of(m * RW, RW)
            copies = []
            for tab, tail, buf, sem in ((guT, tailu, gu_b, sem0),
                                        (giT, taili, gi_b, sem1)):
                for k in range(PACK):
                    if k < PACK - 1:
                        copies.append(pltpu.async_copy(
                            tab.at[:, pl.ds(q0 + k * QS, RW)],
                            buf.at[k], sem))
                    else:
                        @pl.when(q0 + 3 * QS + RW <= U)
                        def _(tab=tab, buf=buf, sem=sem, q0=q0):
                            pltpu.async_copy(
                                tab.at[:, pl.ds(q0 + 3 * QS, RW)],
                                buf.at[3], sem).wait()

                        @pl.when(q0 + 3 * QS == TAILW)
                        def _(tail=tail, buf=buf, sem=sem):
                            pltpu.async_copy(tail, buf.at[3], sem).wait()
            for c in copies:
                c.wait()

            for g in range(RW // L):
                qv = g * L + lax.iota(jnp.int32, L)
                for buf, stage in ((gu_b, gu_o), (gi_b, gi_o)):
                    for k in range(PACK):
                        for c in range(D):
                            col = plsc.load_gather(
                                buf.at[k], [jnp.full((L,), c, jnp.int32), qv])
                            plsc.store_scatter(
                                stage,
                                [qv, jnp.full((L,), 32 * k + c, jnp.int32)],
                                col)
            pltpu.sync_copy(gu_o, pgu.at[pl.ds(q0, RW)])
            pltpu.sync_copy(gi_o, pgi.at[pl.ds(q0, RW)])

        return carry

    lax.fori_loop(0, (NWIN + NW - 1) // NW, win_body, 0)


_mesh = plsc.VectorSubcoreMesh(
    core_axis_name="c", subcore_axis_name="s", num_cores=NC, num_subcores=NS
)


def _quarter(r):
    k = ((r >= QS).astype(jnp.int32) + (r >= 2 * QS).astype(jnp.int32)
         + (r >= 3 * QS).astype(jnp.int32))
    return k


@functools.partial(
    pl.kernel,
    out_type=(
        jax.ShapeDtypeStruct((B,), jnp.float32),
        jax.ShapeDtypeStruct((B // PACK, PACK * D), jnp.float32),
        jax.ShapeDtypeStruct((B // PACK, PACK * D), jnp.float32),
    ),
    mesh=_mesh,
    scratch_types=[
        pltpu.VMEM((BPW,), jnp.int32),
        pltpu.VMEM((BPW,), jnp.int32),
        pltpu.VMEM((BPW,), jnp.int32),
        pltpu.VMEM((BPW,), jnp.int32),
        pltpu.VMEM((2, CH, 128), jnp.float32),
        pltpu.VMEM((2, CH, 128), jnp.float32),
        pltpu.VMEM((2, CH, 128), jnp.float32),
        pltpu.VMEM((2, CH, 128), jnp.float32),
        pltpu.VMEM((D, L), jnp.float32),
        pltpu.VMEM((BPW,), jnp.float32),
        pltpu.VMEM((BPW // PACK, PACK * D), jnp.float32),
        pltpu.VMEM((BPW // PACK, PACK * D), jnp.float32),
        pltpu.SemaphoreType.DMA,
        pltpu.SemaphoreType.DMA,
    ],
    compiler_params=pltpu.CompilerParams(needs_layout_passes=False),
)
def _sc_gather(uidx_hbm, iidx_hbm, gu_hbm, gi_hbm, mu_hbm, mi_hbm, wo_hbm,
               dot_hbm, mlpu_hbm, mlpi_hbm,
               uidx_v, iidx_v, uphys_v, iphys_v,
               gu_b, gi_b, mu_b, mi_b, wo_v, dot_v, mu_c, mi_c,
               sems0, sems1):
    sems = (sems0, sems1)
    wid = lax.axis_index("s") * NC + lax.axis_index("c")
    base = wid * BPW
    pltpu.sync_copy(uidx_hbm.at[pl.ds(base, BPW)], uidx_v)
    pltpu.sync_copy(iidx_hbm.at[pl.ds(base, BPW)], iidx_v)
    pltpu.sync_copy(wo_hbm, wo_v)

    def phys_body(i, carry):
        sl = pl.ds(i * L, L)
        u = uidx_v[sl]
        uphys_v[sl] = u - _quarter(u) * QS
        it = iidx_v[sl]
        iphys_v[sl] = it - _quarter(it) * QS
        return carry

    lax.fori_loop(0, BPW // L, phys_body, 0)

    def start_chunk(j):
        sl = pl.ds(j * CH, CH)
        b = j % 2
        sem = sems[b]
        return [
            pltpu.async_copy(gu_hbm.at[uphys_v.at[sl]], gu_b.at[b], sem),
            pltpu.async_copy(gi_hbm.at[iphys_v.at[sl]], gi_b.at[b], sem),
            pltpu.async_copy(mu_hbm.at[uphys_v.at[sl]], mu_b.at[b], sem),
            pltpu.async_copy(mi_hbm.at[iphys_v.at[sl]], mi_b.at[b], sem),
        ]

    def process_chunk(j):
        b = j % 2
        for g in range(GPC):
            gbase = j * CH + g * L
            rows = g * L + lax.iota(jnp.int32, L)
            grows = gbase + lax.iota(jnp.int32, L)
            uoff = _quarter(uidx_v[pl.ds(gbase, L)]) << 5
            ioff = _quarter(iidx_v[pl.ds(gbase, L)]) << 5

            def col_body(ci, acc, b=b, rows=rows, grows=grows,
                         uoff=uoff, ioff=ioff):
                for u in range(4):
                    c = ci * 4 + u
                    wvec = wo_v[c]
                    ucol = plsc.load_gather(gu_b.at[b], [rows, uoff + c])
                    icol = plsc.load_gather(gi_b.at[b], [rows, ioff + c])
                    acc = acc + ucol * icol * wvec
                    flat = grows * D + c
                    crow = flat >> 7
                    ccol = flat & 127
                    mucol = plsc.load_gather(mu_b.at[b], [rows, uoff + c])
                    plsc.store_scatter(mu_c, [crow, ccol], mucol)
                    micol = plsc.load_gather(mi_b.at[b], [rows, ioff + c])
                    plsc.store_scatter(mi_c, [crow, ccol], micol)
                return acc

            acc = lax.fori_loop(0, D // 4, col_body,
                                jnp.zeros((L,), jnp.float32))
            dot_v[pl.ds(gbase, L)] = acc

    copies = [None] * NCHUNK
    copies[0] = start_chunk(0)
    copies[1] = start_chunk(1)
    for j in range(NCHUNK):
        for c in copies[j]:
            c.wait()
        process_chunk(j)
        if j + 2 < NCHUNK:
            copies[j + 2] = start_chunk(j + 2)

    pltpu.sync_copy(dot_v, dot_hbm.at[pl.ds(base, BPW)])
    pbase = wid * (BPW // PACK)
    pltpu.sync_copy(mu_c, mlpu_hbm.at[pl.ds(pbase, BPW // PACK)])
    pltpu.sync_copy(mi_c, mlpi_hbm.at[pl.ds(pbase, BPW // PACK)])


BM = 512


def _tc_body(dot_ref, mu_ref, mi_ref, w1a_ref, w1b_ref, b1_ref, w2_ref,
             b2_ref, w3_ref, b3_ref, wom_ref, bo_ref, out_ref):
    f32 = jnp.float32
    cols = []
    for k in range(PACK):
        xu = mu_ref[:, D * k:D * k + D]
        xi = mi_ref[:, D * k:D * k + D]
        h = jnp.dot(xu, w1a_ref[...], preferred_element_type=f32)
        h = h + jnp.dot(xi, w1b_ref[...], preferred_element_type=f32)
        h = jnp.maximum(h + b1_ref[...], 0.0)
        h = jnp.maximum(
            jnp.dot(h, w2_ref[...], preferred_element_type=f32)
            + b2_ref[...], 0.0)
        h = jnp.maximum(
            jnp.dot(h, w3_ref[...], preferred_element_type=f32)
            + b3_ref[...], 0.0)
        logit = (jnp.dot(h, wom_ref[...], preferred_element_type=f32)
                 + dot_ref[:, k:k + 1] + bo_ref[...])
        cols.append(1.0 / (1.0 + jnp.exp(-logit)))
    out_ref[...] = jnp.concatenate(cols, axis=1)


_tc_mlp = pl.pallas_call(
    _tc_body,
    grid=(B // PACK // BM,),
    in_specs=[
        pl.BlockSpec((BM, PACK), lambda i: (i, 0)),
        pl.BlockSpec((BM, PACK * D), lambda i: (i, 0)),
        pl.BlockSpec((BM, PACK * D), lambda i: (i, 0)),
        pl.BlockSpec((D, D), lambda i: (0, 0)),
        pl.BlockSpec((D, D), lambda i: (0, 0)),
        pl.BlockSpec((1, D), lambda i: (0, 0)),
        pl.BlockSpec((D, 16), lambda i: (0, 0)),
        pl.BlockSpec((1, 16), lambda i: (0, 0)),
        pl.BlockSpec((16, 8), lambda i: (0, 0)),
        pl.BlockSpec((1, 8), lambda i: (0, 0)),
        pl.BlockSpec((8, 1), lambda i: (0, 0)),
        pl.BlockSpec((1, 1), lambda i: (0, 0)),
    ],
    out_specs=pl.BlockSpec((BM, PACK), lambda i: (i, 0)),
    out_shape=jax.ShapeDtypeStruct((B // PACK, PACK), jnp.float32),
)


def kernel(user_indices, item_indices, gmf_user_table, gmf_item_table,
           mlp_user_table, mlp_item_table, W1, b1, W2, b2, W3, b3, Wo, bo):
    views = []
    for t in (mlp_user_table, mlp_item_table):
        views.extend([t.T] * PACK)
    mu_p, mi_p = _tc_relayout(*views)
    zpad = jnp.zeros((D, RW - (U - TAILW)), jnp.float32)
    gu_p, gi_p = _sc_relayout(
        gmf_user_table.T, gmf_item_table.T,
        jnp.concatenate([gmf_user_table[TAILW:].T, zpad], axis=1),
        jnp.concatenate([gmf_item_table[TAILW:].T, zpad], axis=1))
    wo_gmf = jnp.broadcast_to(Wo[:D], (D, L))
    gmf_dot, mlpu_p, mlpi_p = _sc_gather(
        user_indices, item_indices, gu_p, gi_p, mu_p, mi_p, wo_gmf)
    outp = _tc_mlp(
        gmf_dot.reshape(B // PACK, PACK), mlpu_p, mlpi_p,
        W1[:D], W1[D:], b1.reshape(1, -1),
        W2, b2.reshape(1, -1), W3, b3.reshape(1, -1),
        Wo[D:], bo.reshape(1, 1))
    return outp.reshape(B)


# speedup vs baseline: 2.7083x; 2.7083x over previous
import functools

import jax
import jax.numpy as jnp
from jax import lax
from jax.experimental import pallas as pl
from jax.experimental.pallas import tpu as pltpu
from jax.experimental.pallas import tpu_sc as plsc

B = 16384
D = 32
HD = D // 2
U = 1000000
E8 = 8
TCH = 2048
NB = 62
ES = NB * TCH
NBT = (U + TCH - 1) // TCH - 1
PACK = 4
NC = 2
NS = 16
NW = NC * NS
BPW = B // NW
CH = 32
NCHUNK = BPW // CH
L = 16
GPC = CH // L


def _tr_body(*refs):
    in_refs, out_refs = refs[:32], refs[32:]
    for t in range(4):
        packed = []
        for k in range(E8):
            x = in_refs[E8 * t + k][...]
            u = jax.lax.bitcast_convert_type(
                x.astype(jnp.bfloat16), jnp.uint16).astype(jnp.int32)
            packed.append(u[:HD] | (u[HD:] << 16))
        pieces = [
            jnp.swapaxes(
                jnp.concatenate([packed[2 * k], packed[2 * k + 1]], axis=0),
                0, 1)
            for k in range(E8 // 2)
        ]
        out_refs[t][...] = jnp.concatenate(pieces, axis=1)


def _tr_in_spec(k):
    return pl.BlockSpec(
        (D, TCH), lambda i, k=k: (0, jnp.minimum(k * NB + i, NBT)))


_tc_relayout = pl.pallas_call(
    _tr_body,
    grid=(NB,),
    in_specs=[_tr_in_spec(k) for _ in range(4) for k in range(E8)],
    out_specs=[pl.BlockSpec((TCH, E8 * HD), lambda i: (i, 0))] * 4,
    out_shape=[jax.ShapeDtypeStruct((ES, E8 * HD), jnp.int32)] * 4,
)


_mesh = plsc.VectorSubcoreMesh(
    core_axis_name="c", subcore_axis_name="s", num_cores=NC, num_subcores=NS
)


def _eighth(r):
    k = jnp.zeros(r.shape, jnp.int32)
    for j in range(1, E8):
        k = k + (r >= j * ES).astype(jnp.int32)
    return k


def _unpack_pair(pair):
    lo = jax.lax.bitcast_convert_type(pair << 16, jnp.float32)
    hi = jax.lax.bitcast_convert_type(pair & jnp.int32(-65536), jnp.float32)
    return lo, hi


@functools.partial(
    pl.kernel,
    out_type=(
        jax.ShapeDtypeStruct((B,), jnp.float32),
        jax.ShapeDtypeStruct((B // PACK, PACK * D), jnp.float32),
        jax.ShapeDtypeStruct((B // PACK, PACK * D), jnp.float32),
    ),
    mesh=_mesh,
    scratch_types=[
        pltpu.VMEM((BPW,), jnp.int32),
        pltpu.VMEM((BPW,), jnp.int32),
        pltpu.VMEM((BPW,), jnp.int32),
        pltpu.VMEM((BPW,), jnp.int32),
        pltpu.VMEM((2, CH, E8 * HD), jnp.int32),
        pltpu.VMEM((2, CH, E8 * HD), jnp.int32),
        pltpu.VMEM((2, CH, E8 * HD), jnp.int32),
        pltpu.VMEM((2, CH, E8 * HD), jnp.int32),
        pltpu.VMEM((D, L), jnp.float32),
        pltpu.VMEM((BPW,), jnp.float32),
        pltpu.VMEM((BPW // PACK, PACK * D), jnp.float32),
        pltpu.VMEM((BPW // PACK, PACK * D), jnp.float32),
        pltpu.SemaphoreType.DMA,
        pltpu.SemaphoreType.DMA,
    ],
    compiler_params=pltpu.CompilerParams(needs_layout_passes=False),
)
def _sc_gather(uidx_hbm, iidx_hbm, gu_hbm, gi_hbm, mu_hbm, mi_hbm, wo_hbm,
               dot_hbm, mlpu_hbm, mlpi_hbm,
               uidx_v, iidx_v, uphys_v, iphys_v,
               gu_b, gi_b, mu_b, mi_b, wo_v, dot_v, mu_c, mi_c,
               sems0, sems1):
    sems = (sems0, sems1)
    wid = lax.axis_index("s") * NC + lax.axis_index("c")
    base = wid * BPW
    pltpu.sync_copy(uidx_hbm.at[pl.ds(base, BPW)], uidx_v)
    pltpu.sync_copy(iidx_hbm.at[pl.ds(base, BPW)], iidx_v)
    pltpu.sync_copy(wo_hbm, wo_v)

    def phys_body(i, carry):
        sl = pl.ds(i * L, L)
        u = uidx_v[sl]
        uphys_v[sl] = u - _eighth(u) * ES
        it = iidx_v[sl]
        iphys_v[sl] = it - _eighth(it) * ES
        return carry

    lax.fori_loop(0, BPW // L, phys_body, 0)

    def start_chunk(j):
        sl = pl.ds(j * CH, CH)
        b = j % 2
        sem = sems[b]
        return [
            pltpu.async_copy(gu_hbm.at[uphys_v.at[sl]], gu_b.at[b], sem),
            pltpu.async_copy(gi_hbm.at[iphys_v.at[sl]], gi_b.at[b], sem),
            pltpu.async_copy(mu_hbm.at[uphys_v.at[sl]], mu_b.at[b], sem),
            pltpu.async_copy(mi_hbm.at[iphys_v.at[sl]], mi_b.at[b], sem),
        ]

    def process_chunk(j):
        b = j % 2
        for g in range(GPC):
            gbase = j * CH + g * L
            rows = g * L + lax.iota(jnp.int32, L)
            grows = gbase + lax.iota(jnp.int32, L)
            uoff = _eighth(uidx_v[pl.ds(gbase, L)]) << 4
            ioff = _eighth(iidx_v[pl.ds(gbase, L)]) << 4

            def col_body(ci, acc, b=b, rows=rows, grows=grows,
                         uoff=uoff, ioff=ioff):
                for u in range(2):
                    cp = ci * 2 + u
                    wlo = wo_v[cp]
                    whi = wo_v[cp + HD]
                    upair = plsc.load_gather(gu_b.at[b], [rows, uoff + cp])
                    ipair = plsc.load_gather(gi_b.at[b], [rows, ioff + cp])
                    ulo, uhi = _unpack_pair(upair)
                    ilo, ihi = _unpack_pair(ipair)
                    acc = acc + ulo * ilo * wlo + uhi * ihi * whi
                    flat_lo = grows * D + cp
                    flat_hi = flat_lo + HD
                    mpair = plsc.load_gather(mu_b.at[b], [rows, uoff + cp])
                    mlo, mhi = _unpack_pair(mpair)
                    plsc.store_scatter(
                        mu_c, [flat_lo >> 7, flat_lo & 127], mlo)
                    plsc.store_scatter(
                        mu_c, [flat_hi >> 7, flat_hi & 127], mhi)
                    npair = plsc.load_gather(mi_b.at[b], [rows, ioff + cp])
                    nlo, nhi = _unpack_pair(npair)
                    plsc.store_scatter(
                        mi_c, [flat_lo >> 7, flat_lo & 127], nlo)
                    plsc.store_scatter(
                        mi_c, [flat_hi >> 7, flat_hi & 127], nhi)
                return acc

            acc = lax.fori_loop(0, HD // 2, col_body,
                                jnp.zeros((L,), jnp.float32))
            dot_v[pl.ds(gbase, L)] = acc

    copies = [None] * NCHUNK
    copies[0] = start_chunk(0)
    copies[1] = start_chunk(1)
    for j in range(NCHUNK):
        for c in copies[j]:
            c.wait()
        process_chunk(j)
        if j + 2 < NCHUNK:
            copies[j + 2] = start_chunk(j + 2)

    pltpu.sync_copy(dot_v, dot_hbm.at[pl.ds(base, BPW)])
    pbase = wid * (BPW // PACK)
    pltpu.sync_copy(mu_c, mlpu_hbm.at[pl.ds(pbase, BPW // PACK)])
    pltpu.sync_copy(mi_c, mlpi_hbm.at[pl.ds(pbase, BPW // PACK)])


BM = 512


def _tc_body(dot_ref, mu_ref, mi_ref, w1a_ref, w1b_ref, b1_ref, w2_ref,
             b2_ref, w3_ref, b3_ref, wom_ref, bo_ref, out_ref):
    f32 = jnp.float32
    cols = []
    for k in range(PACK):
        xu = mu_ref[:, D * k:D * k + D]
        xi = mi_ref[:, D * k:D * k + D]
        h = jnp.dot(xu, w1a_ref[...], preferred_element_type=f32)
        h = h + jnp.dot(xi, w1b_ref[...], preferred_element_type=f32)
        h = jnp.maximum(h + b1_ref[...], 0.0)
        h = jnp.maximum(
            jnp.dot(h, w2_ref[...], preferred_element_type=f32)
            + b2_ref[...], 0.0)
        h = jnp.maximum(
            jnp.dot(h, w3_ref[...], preferred_element_type=f32)
            + b3_ref[...], 0.0)
        logit = (jnp.dot(h, wom_ref[...], preferred_element_type=f32)
                 + dot_ref[:, k:k + 1] + bo_ref[...])
        cols.append(1.0 / (1.0 + jnp.exp(-logit)))
    out_ref[...] = jnp.concatenate(cols, axis=1)


_tc_mlp = pl.pallas_call(
    _tc_body,
    grid=(B // PACK // BM,),
    in_specs=[
        pl.BlockSpec((BM, PACK), lambda i: (i, 0)),
        pl.BlockSpec((BM, PACK * D), lambda i: (i, 0)),
        pl.BlockSpec((BM, PACK * D), lambda i: (i, 0)),
        pl.BlockSpec((D, D), lambda i: (0, 0)),
        pl.BlockSpec((D, D), lambda i: (0, 0)),
        pl.BlockSpec((1, D), lambda i: (0, 0)),
        pl.BlockSpec((D, 16), lambda i: (0, 0)),
        pl.BlockSpec((1, 16), lambda i: (0, 0)),
        pl.BlockSpec((16, 8), lambda i: (0, 0)),
        pl.BlockSpec((1, 8), lambda i: (0, 0)),
        pl.BlockSpec((8, 1), lambda i: (0, 0)),
        pl.BlockSpec((1, 1), lambda i: (0, 0)),
    ],
    out_specs=pl.BlockSpec((BM, PACK), lambda i: (i, 0)),
    out_shape=jax.ShapeDtypeStruct((B // PACK, PACK), jnp.float32),
)


def kernel(user_indices, item_indices, gmf_user_table, gmf_item_table,
           mlp_user_table, mlp_item_table, W1, b1, W2, b2, W3, b3, Wo, bo):
    views = []
    for t in (gmf_user_table, gmf_item_table, mlp_user_table, mlp_item_table):
        views.extend([t.T] * E8)
    gu_p, gi_p, mu_p, mi_p = _tc_relayout(*views)
    wo_gmf = jnp.broadcast_to(Wo[:D], (D, L))
    gmf_dot, mlpu_p, mlpi_p = _sc_gather(
        user_indices, item_indices, gu_p, gi_p, mu_p, mi_p, wo_gmf)
    outp = _tc_mlp(
        gmf_dot.reshape(B // PACK, PACK), mlpu_p, mlpi_p,
        W1[:D], W1[D:], b1.reshape(1, -1),
        W2, b2.reshape(1, -1), W3, b3.reshape(1, -1),
        Wo[D:], bo.reshape(1, 1))
    return outp.reshape(B)
